# baseline (speedup 1.0000x reference)
import functools

import jax
import jax.numpy as jnp
import numpy as np
from jax import lax
from jax.experimental import pallas as pl
from jax.experimental.pallas import tpu as pltpu
from jax.experimental.pallas import tpu_sc as plsc

D_VEC = 128
L_POS = 50
K_SET = 16
N_CH = 4096
HID = 256

_NEG = float(np.finfo(np.float32).min)
_SEL_C = 256
_MLP_C = 2048
_NCORE = 2
_NW = 32


def _select_body(s_ref, i_ref, g_ref, n_ref):
    ms = s_ref[...]
    c = ms.shape[1]
    jio = lax.broadcasted_iota(jnp.int32, (L_POS, c), 0)
    rank = jnp.zeros((L_POS, c), jnp.float32)
    for i in range(L_POS):
        ri = ms[i:i + 1, :]
        cmp = (ri > ms) | ((ri >= ms) & (i < jio))
        rank = rank + jnp.where(cmp, 1.0, 0.0)
    sel = (ms > -1e37) & (rank < float(K_SET))
    self_ = jnp.where(sel, 1.0, 0.0)
    a = lax.broadcasted_iota(jnp.int32, (L_POS, L_POS), 0)
    b = lax.broadcasted_iota(jnp.int32, (L_POS, L_POS), 1)
    tri = jnp.where(a > b, 1.0, 0.0)
    slot = jnp.dot(tri, self_, preferred_element_type=jnp.float32)
    n_ref[...] = slot[L_POS - 1:L_POS, :] + self_[L_POS - 1:L_POS, :]
    idxf = i_ref[...]
    for s in range(K_SET):
        on = self_ * jnp.where(slot == float(s), 1.0, 0.0)
        g_ref[s:s + 1, :] = jnp.sum(idxf * on, axis=0,
                                    keepdims=True).astype(jnp.int32)


def _select(msT, iT):
    grid = N_CH // _SEL_C
    return pl.pallas_call(
        _select_body,
        grid=(grid,),
        in_specs=[pl.BlockSpec((L_POS, _SEL_C), lambda b: (0, b))
                  for _ in range(2)],
        out_specs=[pl.BlockSpec((K_SET, _SEL_C), lambda b: (0, b)),
                   pl.BlockSpec((1, _SEL_C), lambda b: (0, b))],
        out_shape=[jax.ShapeDtypeStruct((K_SET, N_CH), jnp.int32),
                   jax.ShapeDtypeStruct((1, N_CH), jnp.float32)],
    )(msT, iT)


def _gather(gT, v, h, H):
    mesh = plsc.VectorSubcoreMesh(core_axis_name="c", subcore_axis_name="s")
    rows_per_w = (H * K_SET) // _NW // 128

    @functools.partial(
        pl.kernel, mesh=mesh,
        out_type=jax.ShapeDtypeStruct((H, K_SET * D_VEC), jnp.float32),
        scratch_types=[
            pltpu.VMEM((rows_per_w * 128,), jnp.int32),
            pltpu.VMEM((128, D_VEC), jnp.float32),
            pltpu.VMEM((128, D_VEC), jnp.float32),
            pltpu.VMEM((128, D_VEC), jnp.float32),
            pltpu.VMEM((128, D_VEC), jnp.float32),
            pltpu.SemaphoreType.DMA,
            pltpu.SemaphoreType.DMA,
            pltpu.SemaphoreType.DMA,
            pltpu.SemaphoreType.DMA,
            pltpu.SemaphoreType.DMA,
            pltpu.SemaphoreType.DMA,
            pltpu.SemaphoreType.DMA,
            pltpu.SemaphoreType.DMA,
        ])
    def k(idx_hbm, table_hbm, out_hbm, idx_v,
          b0, b1, b2, b3, g0, g1, g2, g3, w0, w1, w2, w3):
        wid = lax.axis_index("s") * _NCORE + lax.axis_index("c")
        s_slot = wid // 2
        c_base = (wid % 2) * (H // 2)
        pltpu.sync_copy(
            idx_hbm.at[s_slot, pl.ds(h * H + c_base, rows_per_w * 128)],
            idx_v)
        bufs = (b0, b1, b2, b3)
        gsems = (g0, g1, g2, g3)
        wsems = (w0, w1, w2, w3)
        nb = 4
        gcp = [None] * nb
        wcp = [None] * nb
        for j in range(rows_per_w + 1):
            if j < rows_per_w:
                b = j % nb
                if wcp[b] is not None:
                    wcp[b].wait()
                gcp[b] = pltpu.async_copy(
                    table_hbm.at[idx_v.at[pl.ds(j * 128, 128)]],
                    bufs[b], gsems[b])
            if j >= 1:
                b = (j - 1) % nb
                gcp[b].wait()
                wcp[b] = pltpu.async_copy(
                    bufs[b],
                    out_hbm.at[pl.ds(c_base + (j - 1) * 128, 128),
                               pl.ds(s_slot * D_VEC, D_VEC)],
                    wsems[b])
        for b in range(nb):
            if wcp[b] is not None:
                wcp[b].wait()

    return k(gT, v)


def _mlp_body(p_ref, n_ref, c_ref, w1_ref, wc_ref, b1_ref, w2_ref, b2_ref,
              o_ref):
    ns = n_ref[...]
    si = lax.broadcasted_iota(jnp.int32, (_MLP_C, K_SET * D_VEC), 1) // D_VEC
    x = p_ref[...] * jnp.where(si < ns, 1.0, 0.0)
    h = jnp.dot(x, w1_ref[...], preferred_element_type=jnp.float32)
    h = h + jnp.log1p(c_ref[...]) * wc_ref[0:1, :] + b1_ref[...]
    act = 0.5 * h * (1.0 + lax.erf(h * np.float32(1.0 / np.sqrt(2.0))))
    o_ref[...] = (jnp.dot(act, w2_ref[...],
                          preferred_element_type=jnp.float32) + b2_ref[...])


def _mlp(packed, nsel_c, cnt_c, W1, b1r, W2, b2r):
    n_rows = packed.shape[0]
    grid = n_rows // _MLP_C
    return pl.pallas_call(
        _mlp_body,
        grid=(grid,),
        in_specs=[
            pl.BlockSpec((_MLP_C, K_SET * D_VEC), lambda b: (b, 0)),
            pl.BlockSpec((_MLP_C, 1), lambda b: (b, 0)),
            pl.BlockSpec((_MLP_C, 1), lambda b: (b, 0)),
            pl.BlockSpec((K_SET * D_VEC, HID), lambda b: (0, 0)),
            pl.BlockSpec((8, HID), lambda b: (K_SET * D_VEC // 8, 0)),
            pl.BlockSpec((1, HID), lambda b: (0, 0)),
            pl.BlockSpec((HID, D_VEC), lambda b: (0, 0)),
            pl.BlockSpec((1, D_VEC), lambda b: (0, 0)),
        ],
        out_specs=pl.BlockSpec((_MLP_C, D_VEC), lambda b: (b, 0)),
        out_shape=jax.ShapeDtypeStruct((n_rows, D_VEC), jnp.float32),
    )(packed, nsel_c, cnt_c, W1, W1, b1r, W2, b2r)


_SPLIT = 1


def kernel(v, batch_idx, mask, count, rank_scores, W1, b1, W2, b2):
    msT = jnp.where(mask, rank_scores, _NEG).T
    iT = batch_idx.astype(jnp.float32).T
    gT, nselT = _select(msT, iT)
    nsel_c = nselT.reshape(N_CH, 1)
    cnt_c = count.reshape(N_CH, 1)
    b1r = b1.reshape(1, HID)
    b2r = b2.reshape(1, D_VEC)
    H = N_CH // _SPLIT
    outs = []
    for h in range(_SPLIT):
        packed_h = _gather(gT, v, h, H)
        outs.append(_mlp(packed_h, nsel_c[h * H:(h + 1) * H],
                         cnt_c[h * H:(h + 1) * H], W1, b1r, W2, b2r))
    return outs[0] if _SPLIT == 1 else jnp.concatenate(outs, axis=0)


# speedup vs baseline: 1.0157x; 1.0157x over previous
import functools

import jax
import jax.numpy as jnp
import numpy as np
from jax import lax
from jax.experimental import pallas as pl
from jax.experimental.pallas import tpu as pltpu
from jax.experimental.pallas import tpu_sc as plsc

D_VEC = 128
L_POS = 50
K_SET = 16
N_CH = 4096
HID = 256

_NEG = float(np.finfo(np.float32).min)
_SEL_C = 256
_MLP_C = 1024
_NCORE = 2
_NW = 32


def _select_body(s_ref, i_ref, g_ref, n_ref):
    ms = s_ref[...]
    c = ms.shape[1]
    jio = lax.broadcasted_iota(jnp.int32, (L_POS, c), 0)
    rank = jnp.zeros((L_POS, c), jnp.float32)
    for i in range(L_POS):
        ri = ms[i:i + 1, :]
        cmp = (ri > ms) | ((ri >= ms) & (i < jio))
        rank = rank + jnp.where(cmp, 1.0, 0.0)
    sel = (ms > -1e37) & (rank < float(K_SET))
    self_ = jnp.where(sel, 1.0, 0.0)
    a = lax.broadcasted_iota(jnp.int32, (L_POS, L_POS), 0)
    b = lax.broadcasted_iota(jnp.int32, (L_POS, L_POS), 1)
    tri = jnp.where(a > b, 1.0, 0.0)
    slot = jnp.dot(tri, self_, preferred_element_type=jnp.float32)
    n_ref[...] = slot[L_POS - 1:L_POS, :] + self_[L_POS - 1:L_POS, :]
    idxf = i_ref[...]
    for s in range(K_SET):
        on = self_ * jnp.where(slot == float(s), 1.0, 0.0)
        g_ref[s:s + 1, :] = jnp.sum(idxf * on, axis=0,
                                    keepdims=True).astype(jnp.int32)


def _select(msT, iT):
    grid = N_CH // _SEL_C
    return pl.pallas_call(
        _select_body,
        grid=(grid,),
        in_specs=[pl.BlockSpec((L_POS, _SEL_C), lambda b: (0, b))
                  for _ in range(2)],
        out_specs=[pl.BlockSpec((K_SET, _SEL_C), lambda b: (0, b)),
                   pl.BlockSpec((1, _SEL_C), lambda b: (0, b))],
        out_shape=[jax.ShapeDtypeStruct((K_SET, N_CH), jnp.int32),
                   jax.ShapeDtypeStruct((1, N_CH), jnp.float32)],
    )(msT, iT)


def _gather(gT, v, h, H):
    mesh = plsc.VectorSubcoreMesh(core_axis_name="c", subcore_axis_name="s")
    rows_per_w = (H * K_SET) // _NW // 128

    @functools.partial(
        pl.kernel, mesh=mesh,
        out_type=jax.ShapeDtypeStruct((H, K_SET * D_VEC), jnp.float32),
        scratch_types=[
            pltpu.VMEM((rows_per_w * 128,), jnp.int32),
            pltpu.VMEM((128, D_VEC), jnp.float32),
            pltpu.VMEM((128, D_VEC), jnp.float32),
            pltpu.VMEM((128, D_VEC), jnp.float32),
            pltpu.VMEM((128, D_VEC), jnp.float32),
            pltpu.SemaphoreType.DMA,
            pltpu.SemaphoreType.DMA,
            pltpu.SemaphoreType.DMA,
            pltpu.SemaphoreType.DMA,
            pltpu.SemaphoreType.DMA,
            pltpu.SemaphoreType.DMA,
            pltpu.SemaphoreType.DMA,
            pltpu.SemaphoreType.DMA,
        ])
    def k(idx_hbm, table_hbm, out_hbm, idx_v,
          b0, b1, b2, b3, g0, g1, g2, g3, w0, w1, w2, w3):
        wid = lax.axis_index("s") * _NCORE + lax.axis_index("c")
        s_slot = wid // 2
        c_base = (wid % 2) * (H // 2)
        pltpu.sync_copy(
            idx_hbm.at[s_slot, pl.ds(h * H + c_base, rows_per_w * 128)],
            idx_v)
        bufs = (b0, b1, b2, b3)
        gsems = (g0, g1, g2, g3)
        wsems = (w0, w1, w2, w3)
        nb = 4
        gcp = [None] * nb
        wcp = [None] * nb
        for j in range(rows_per_w + 1):
            if j < rows_per_w:
                b = j % nb
                if wcp[b] is not None:
                    wcp[b].wait()
                gcp[b] = pltpu.async_copy(
                    table_hbm.at[idx_v.at[pl.ds(j * 128, 128)]],
                    bufs[b], gsems[b])
            if j >= 1:
                b = (j - 1) % nb
                gcp[b].wait()
                wcp[b] = pltpu.async_copy(
                    bufs[b],
                    out_hbm.at[pl.ds(c_base + (j - 1) * 128, 128),
                               pl.ds(s_slot * D_VEC, D_VEC)],
                    wsems[b])
        for b in range(nb):
            if wcp[b] is not None:
                wcp[b].wait()

    return k(gT, v)


def _mlp_body(p_ref, n_ref, c_ref, w1_ref, wc_ref, b1_ref, w2_ref, b2_ref,
              o_ref):
    ns = n_ref[...]
    si = lax.broadcasted_iota(jnp.int32, (_MLP_C, K_SET * D_VEC), 1) // D_VEC
    x = p_ref[...] * jnp.where(si < ns, 1.0, 0.0)
    h = jnp.dot(x, w1_ref[...], preferred_element_type=jnp.float32)
    h = h + jnp.log1p(c_ref[...]) * wc_ref[0:1, :] + b1_ref[...]
    act = 0.5 * h * (1.0 + lax.erf(h * np.float32(1.0 / np.sqrt(2.0))))
    o_ref[...] = (jnp.dot(act, w2_ref[...],
                          preferred_element_type=jnp.float32) + b2_ref[...])


def _mlp(packed, nsel_c, cnt_c, W1, b1r, W2, b2r):
    n_rows = packed.shape[0]
    grid = n_rows // _MLP_C
    return pl.pallas_call(
        _mlp_body,
        grid=(grid,),
        in_specs=[
            pl.BlockSpec((_MLP_C, K_SET * D_VEC), lambda b: (b, 0)),
            pl.BlockSpec((_MLP_C, 1), lambda b: (b, 0)),
            pl.BlockSpec((_MLP_C, 1), lambda b: (b, 0)),
            pl.BlockSpec((K_SET * D_VEC, HID), lambda b: (0, 0)),
            pl.BlockSpec((8, HID), lambda b: (K_SET * D_VEC // 8, 0)),
            pl.BlockSpec((1, HID), lambda b: (0, 0)),
            pl.BlockSpec((HID, D_VEC), lambda b: (0, 0)),
            pl.BlockSpec((1, D_VEC), lambda b: (0, 0)),
        ],
        out_specs=pl.BlockSpec((_MLP_C, D_VEC), lambda b: (b, 0)),
        out_shape=jax.ShapeDtypeStruct((n_rows, D_VEC), jnp.float32),
    )(packed, nsel_c, cnt_c, W1, W1, b1r, W2, b2r)


_SPLIT = 1


def kernel(v, batch_idx, mask, count, rank_scores, W1, b1, W2, b2):
    msT = jnp.where(mask, rank_scores, _NEG).T
    iT = batch_idx.astype(jnp.float32).T
    gT, nselT = _select(msT, iT)
    nsel_c = nselT.reshape(N_CH, 1)
    cnt_c = count.reshape(N_CH, 1)
    b1r = b1.reshape(1, HID)
    b2r = b2.reshape(1, D_VEC)
    H = N_CH // _SPLIT
    outs = []
    for h in range(_SPLIT):
        packed_h = _gather(gT, v, h, H)
        outs.append(_mlp(packed_h, nsel_c[h * H:(h + 1) * H],
                         cnt_c[h * H:(h + 1) * H], W1, b1r, W2, b2r))
    return outs[0] if _SPLIT == 1 else jnp.concatenate(outs, axis=0)
